# Initial kernel scaffold; baseline (speedup 1.0000x reference)
import jax
import jax.numpy as jnp
from jax.experimental import pallas as pl


def kernel(x, token_table, pos_table):
    raise NotImplementedError("write your pallas kernel here")


# speedup vs baseline: 1.3899x; 1.3899x over previous
import functools

import jax
import jax.numpy as jnp
from jax import lax
from jax.experimental import pallas as pl
from jax.experimental.pallas import tpu as pltpu
from jax.experimental.pallas import tpu_sc as plsc

VOCAB = 1000000
MAXLEN = 200
EMBED = 32
BATCH = 4096

NC = 2
NS = 16
NW = NC * NS

T = BATCH * MAXLEN
TPW = T // NW
R = 4
C = R * MAXLEN
NCHUNK = TPW // C
SUB = 100
SPC = C // SUB
IDX_ROWS_PER_W = TPW // SUB


def _make_kernel():
    mesh = plsc.VectorSubcoreMesh(core_axis_name="c", subcore_axis_name="s")

    @functools.partial(
        pl.kernel,
        out_type=jax.ShapeDtypeStruct((T, EMBED), jnp.float32),
        mesh=mesh,
        scratch_types=[
            pltpu.VMEM((IDX_ROWS_PER_W, SUB), jnp.int32),
            pltpu.VMEM((C, EMBED), jnp.float32),
            pltpu.VMEM((MAXLEN, EMBED), jnp.float32),
            pltpu.SemaphoreType.DMA,
        ],
        compiler_params=pltpu.CompilerParams(use_tc_tiling_on_sc=False),
    )
    def emb(x_hbm, tok_hbm, pos_hbm, out_hbm, idx_v, rows_v, pos_v, sem):
        cid = lax.axis_index("c")
        sid = lax.axis_index("s")
        wid = sid * NC + cid

        pltpu.sync_copy(pos_hbm, pos_v)
        pltpu.sync_copy(
            x_hbm.at[pl.ds(wid * IDX_ROWS_PER_W, IDX_ROWS_PER_W)], idx_v)

        def do_chunk(g, carry):
            copies = []
            for si in range(SPC):
                row = g * SPC + si
                copies.append(pltpu.async_copy(
                    tok_hbm.at[idx_v.at[row]],
                    rows_v.at[pl.ds(si * SUB, SUB)],
                    sem))
            for cp in copies:
                cp.wait()

            def add_l(l, c2):
                p0 = pos_v[l, pl.ds(0, 16)]
                p1 = pos_v[l, pl.ds(16, 16)]
                for r in range(R):
                    t = r * MAXLEN + l
                    rows_v[t, pl.ds(0, 16)] += p0
                    rows_v[t, pl.ds(16, 16)] += p1
                return c2

            lax.fori_loop(0, MAXLEN, add_l, 0, unroll=2)

            pltpu.sync_copy(
                rows_v, out_hbm.at[pl.ds(wid * TPW + g * C, C)])
            return carry

        lax.fori_loop(0, NCHUNK, do_chunk, 0)

    return emb


_emb = _make_kernel()


def kernel(x, token_table, pos_table):
    b, l = x.shape
    x2 = x.reshape(T // SUB, SUB).astype(jnp.int32)
    out = _emb(x2, token_table, pos_table)
    return out.reshape(b, l, EMBED)
